# Initial kernel scaffold; baseline (speedup 1.0000x reference)
import jax
import jax.numpy as jnp
from jax.experimental import pallas as pl


def kernel(x, edge_index, rel):
    raise NotImplementedError("write your pallas kernel here")


# speedup vs baseline: 2.3312x; 2.3312x over previous
import functools

import jax
import jax.numpy as jnp
from jax import lax
from jax.experimental import pallas as pl
from jax.experimental.pallas import tpu as pltpu
from jax.experimental.pallas import tpu_sc as plsc

_N_EDGES = 65536
_D = 1024
_NC = 2
_NS = 16
_NW = _NC * _NS
_EPW = _N_EDGES // _NW
_C = 16
_NCHUNK = _EPW // _C
_NSLICE = _D // 16


def _body(x_hbm, si_hbm, di_hbm, rel_hbm, out_hbm,
          src_idx, dst_idx, rel_v, head0, tail0, head1, tail1,
          scores_v, sem0, sem1):
    wid = lax.axis_index("s") * _NC + lax.axis_index("c")

    pltpu.sync_copy(si_hbm.at[wid], src_idx)
    pltpu.sync_copy(di_hbm.at[wid], dst_idx)
    pltpu.sync_copy(rel_hbm, rel_v)

    def start(c, head, tail, sem):
        pltpu.make_async_copy(x_hbm.at[src_idx.at[c]], head, sem).start()
        pltpu.make_async_copy(x_hbm.at[dst_idx.at[c]], tail, sem).start()

    def drain(head, tail, sem):
        pltpu.make_async_copy(x_hbm.at[src_idx.at[0]], head, sem).wait()
        pltpu.make_async_copy(x_hbm.at[dst_idx.at[0]], tail, sem).wait()

    iota = lax.iota(jnp.int32, 16)

    def compute(c, head, tail):
        def sstep(s, accs):
            r = rel_v[pl.ds(s * 16, 16)]
            return tuple(
                accs[e] + head[e, pl.ds(s * 16, 16)] * r * tail[e, pl.ds(s * 16, 16)]
                for e in range(_C)
            )
        accs = lax.fori_loop(
            0, _NSLICE, sstep,
            tuple(jnp.zeros((16,), jnp.float32) for _ in range(_C)))
        tot = jnp.zeros((16,), jnp.float32)
        for e in range(_C):
            tot = jnp.where(iota == e, jnp.sum(accs[e]), tot)
        scores_v[pl.ds(c * _C, _C)] = tot

    start(0, head0, tail0, sem0)

    def step(k, carry):
        c0 = 2 * k
        start(c0 + 1, head1, tail1, sem1)
        drain(head0, tail0, sem0)
        compute(c0, head0, tail0)

        @pl.when(k < _NCHUNK // 2 - 1)
        def _():
            start(c0 + 2, head0, tail0, sem0)

        drain(head1, tail1, sem1)
        compute(c0 + 1, head1, tail1)
        return carry

    lax.fori_loop(0, _NCHUNK // 2, step, 0)

    pltpu.sync_copy(scores_v, out_hbm.at[pl.ds(wid * _EPW, _EPW)])


@jax.jit
def kernel(x, edge_index, rel):
    ei = edge_index.astype(jnp.int32).reshape(2, _NW, _NCHUNK, _C)
    mesh = plsc.VectorSubcoreMesh(
        core_axis_name="c", subcore_axis_name="s",
        num_cores=_NC, num_subcores=_NS)
    f = pl.kernel(
        _body,
        out_type=jax.ShapeDtypeStruct((_N_EDGES,), jnp.float32),
        mesh=mesh,
        compiler_params=pltpu.CompilerParams(needs_layout_passes=False),
        scratch_types=[
            pltpu.VMEM((_NCHUNK, _C), jnp.int32),
            pltpu.VMEM((_NCHUNK, _C), jnp.int32),
            pltpu.VMEM((_D,), jnp.float32),
            pltpu.VMEM((_C, _D), jnp.float32),
            pltpu.VMEM((_C, _D), jnp.float32),
            pltpu.VMEM((_C, _D), jnp.float32),
            pltpu.VMEM((_C, _D), jnp.float32),
            pltpu.VMEM((_EPW,), jnp.float32),
            pltpu.SemaphoreType.DMA,
            pltpu.SemaphoreType.DMA,
        ],
    )
    return f(x, ei[0], ei[1], rel)
